# baseline (speedup 1.0000x reference)
import functools

import jax
import jax.numpy as jnp
from jax import lax
from jax.experimental import pallas as pl
from jax.experimental.pallas import tpu as pltpu
from jax.experimental.pallas import tpu_sc as plsc

LANES = 16


def _make_tc_matvec(V, E, block_v=16384):
    grid = (V + block_v - 1) // block_v

    def body(tT_ref, w_ref, t_ref):
        w = w_ref[...]
        blk = tT_ref[...]
        t_ref[...] = jnp.sum(blk * w, axis=0)

    return pl.pallas_call(
        body,
        grid=(grid,),
        in_specs=[
            pl.BlockSpec((E, block_v), lambda i: (0, i)),
            pl.BlockSpec((E, 1), lambda i: (0, 0)),
        ],
        out_specs=pl.BlockSpec((block_v,), lambda i: (i,)),
        out_shape=jax.ShapeDtypeStruct((V,), jnp.float32),
    )


def _make_sc_kernel(B, L, V, num_cores, num_subcores, rows_per_chunk=16):
    NW = num_cores * num_subcores
    assert B % NW == 0, (B, NW)
    rows_per_w = B // NW
    toks_per_w = rows_per_w * L
    assert rows_per_w % (2 * rows_per_chunk) == 0
    chunk = rows_per_chunk * L
    n_chunks = toks_per_w // chunk
    assert chunk % 8 == 0 and toks_per_w % 8 == 0

    nfull = L // LANES
    rem = L - nfull * LANES

    mesh = plsc.VectorSubcoreMesh(core_axis_name="c", subcore_axis_name="s")

    @functools.partial(
        pl.kernel,
        out_type=jax.ShapeDtypeStruct((B,), jnp.float32),
        mesh=mesh,
        compiler_params=pltpu.CompilerParams(
            needs_layout_passes=False, use_tc_tiling_on_sc=False),
        scratch_types=[
            pltpu.VMEM((toks_per_w,), jnp.int32),
            pltpu.VMEM((chunk,), jnp.float32),
            pltpu.VMEM((chunk,), jnp.float32),
            pltpu.VMEM((16,), jnp.float32),
            pltpu.VMEM((rows_per_w,), jnp.float32),
            pltpu.SemaphoreType.DMA,
            pltpu.SemaphoreType.DMA,
        ],
    )
    def sc_kernel(x_hbm, bias_hbm, t_hbm, out_hbm,
                  idx_v, vals_a, vals_b, bias_v, out_v, sem_a, sem_b):
        wid = lax.axis_index("s") * num_cores + lax.axis_index("c")

        tok_base = pl.multiple_of(wid * toks_per_w, 8)
        pltpu.sync_copy(x_hbm.at[pl.ds(tok_base, toks_per_w)], idx_v)
        pltpu.sync_copy(bias_hbm, bias_v)

        bias = bias_v[pl.ds(0, LANES)][0]
        inv_l = jnp.float32(1.0 / L)
        lane = lax.iota(jnp.int32, LANES)
        m_lo = (lane < rem).astype(jnp.float32)
        m_hi = jnp.float32(1.0) - m_lo

        def gather(c, buf, sem):
            off = pl.multiple_of(c * chunk, 8)
            return pltpu.async_copy(t_hbm.at[idx_v.at[pl.ds(off, chunk)]],
                                    buf, sem)

        def wait(buf, sem):
            pltpu.make_async_copy(t_hbm.at[idx_v.at[pl.ds(0, chunk)]],
                                  buf, sem).wait()

        def accumulate(c, buf):
            row0 = c * rows_per_chunk
            for p in range(rows_per_chunk // 2):
                base = p * 2 * L
                acc_a = buf[pl.ds(base, LANES)]
                for k in range(1, nfull):
                    acc_a = acc_a + buf[pl.ds(base + k * LANES, LANES)]
                vm = buf[pl.ds(base + nfull * LANES, LANES)]
                acc_b = buf[pl.ds(base + L + rem, LANES)]
                for k in range(1, nfull):
                    acc_b = acc_b + buf[pl.ds(base + L + rem + k * LANES,
                                              LANES)]
                s0 = jnp.sum(acc_a + vm * m_lo) * inv_l + bias
                s1 = jnp.sum(acc_b + vm * m_hi) * inv_l + bias
                row = row0 + 2 * p
                sv = jnp.where(lane == 0, s0, s1)
                plsc.store_scatter(
                    out_v, [jnp.where(lane == 0, row, row + 1)], sv,
                    mask=lane < 2)

        gather(0, vals_a, sem_a)
        gather(1, vals_b, sem_b)

        def body(j, _):
            c0 = 2 * j
            wait(vals_a, sem_a)

            @pl.when(c0 + 2 < n_chunks)
            def _():
                gather(c0 + 2, vals_a, sem_a)

            accumulate(c0, vals_a)

            wait(vals_b, sem_b)

            @pl.when(c0 + 3 < n_chunks)
            def _():
                gather(c0 + 3, vals_b, sem_b)

            accumulate(c0 + 1, vals_b)
            return 0

        lax.fori_loop(0, n_chunks // 2, body, 0)

        out_base = pl.multiple_of(wid * rows_per_w, 8)
        pltpu.sync_copy(out_v, out_hbm.at[pl.ds(out_base, rows_per_w)])

    return sc_kernel


def kernel(x, table, W, b):
    B, L = x.shape
    V, E = table.shape
    info = plsc.get_sparse_core_info()

    t = _make_tc_matvec(V, E)(table.T, W)

    sc = _make_sc_kernel(B, L, V, info.num_cores, info.num_subcores)
    x_flat = x.reshape(-1).astype(jnp.int32)
    bias_vec = jnp.pad(b.reshape(-1).astype(jnp.float32), (0, 15))
    out = sc(x_flat, bias_vec, t)
    return out.reshape(B, 1)


# speedup vs baseline: 1.2191x; 1.2191x over previous
import functools

import jax
import jax.numpy as jnp
from jax import lax
from jax.experimental import pallas as pl
from jax.experimental.pallas import tpu as pltpu
from jax.experimental.pallas import tpu_sc as plsc

LANES = 16


def _make_tc_matvec(V, E, block_v=65536):
    grid = (V + block_v - 1) // block_v

    def body(tT_ref, w_ref, t_ref):
        w = w_ref[...]
        blk = tT_ref[...]
        t_ref[...] = jnp.sum(blk * w, axis=0)

    return pl.pallas_call(
        body,
        grid=(grid,),
        in_specs=[
            pl.BlockSpec((E, block_v), lambda i: (0, i)),
            pl.BlockSpec((E, 1), lambda i: (0, 0)),
        ],
        out_specs=pl.BlockSpec((block_v,), lambda i: (i,)),
        out_shape=jax.ShapeDtypeStruct((V,), jnp.float32),
    )


def _make_sc_kernel(B, L, V, num_cores, num_subcores, rows_per_chunk=16):
    NW = num_cores * num_subcores
    assert B % NW == 0, (B, NW)
    rows_per_w = B // NW
    toks_per_w = rows_per_w * L
    assert rows_per_w % (2 * rows_per_chunk) == 0
    chunk = rows_per_chunk * L
    n_chunks = toks_per_w // chunk
    assert chunk % 8 == 0 and toks_per_w % 8 == 0

    nfull = L // LANES
    rem = L - nfull * LANES

    mesh = plsc.VectorSubcoreMesh(core_axis_name="c", subcore_axis_name="s")

    @functools.partial(
        pl.kernel,
        out_type=jax.ShapeDtypeStruct((B,), jnp.float32),
        mesh=mesh,
        compiler_params=pltpu.CompilerParams(
            needs_layout_passes=False, use_tc_tiling_on_sc=False),
        scratch_types=[
            pltpu.VMEM((toks_per_w,), jnp.int32),
            pltpu.VMEM((chunk,), jnp.float32),
            pltpu.VMEM((chunk,), jnp.float32),
            pltpu.VMEM((16,), jnp.float32),
            pltpu.VMEM((rows_per_w,), jnp.float32),
            pltpu.SemaphoreType.DMA,
            pltpu.SemaphoreType.DMA,
        ],
    )
    def sc_kernel(x_hbm, bias_hbm, t_hbm, out_hbm,
                  idx_v, vals_a, vals_b, bias_v, out_v, sem_a, sem_b):
        wid = lax.axis_index("s") * num_cores + lax.axis_index("c")

        tok_base = pl.multiple_of(wid * toks_per_w, 8)
        pltpu.sync_copy(x_hbm.at[pl.ds(tok_base, toks_per_w)], idx_v)
        pltpu.sync_copy(bias_hbm, bias_v)

        bias = bias_v[pl.ds(0, LANES)][0]
        inv_l = jnp.float32(1.0 / L)
        lane = lax.iota(jnp.int32, LANES)
        m_lo = (lane < rem).astype(jnp.float32)
        m_hi = jnp.float32(1.0) - m_lo

        def gather(c, buf, sem):
            off = pl.multiple_of(c * chunk, 8)
            return pltpu.async_copy(t_hbm.at[idx_v.at[pl.ds(off, chunk)]],
                                    buf, sem)

        def wait(buf, sem):
            pltpu.make_async_copy(t_hbm.at[idx_v.at[pl.ds(0, chunk)]],
                                  buf, sem).wait()

        def accumulate(c, buf):
            row0 = c * rows_per_chunk
            for p in range(rows_per_chunk // 2):
                base = p * 2 * L
                acc_a = buf[pl.ds(base, LANES)]
                for k in range(1, nfull):
                    acc_a = acc_a + buf[pl.ds(base + k * LANES, LANES)]
                vm = buf[pl.ds(base + nfull * LANES, LANES)]
                acc_b = buf[pl.ds(base + L + rem, LANES)]
                for k in range(1, nfull):
                    acc_b = acc_b + buf[pl.ds(base + L + rem + k * LANES,
                                              LANES)]
                s0 = jnp.sum(acc_a + vm * m_lo) * inv_l + bias
                s1 = jnp.sum(acc_b + vm * m_hi) * inv_l + bias
                row = row0 + 2 * p
                sv = jnp.where(lane == 0, s0, s1)
                plsc.store_scatter(
                    out_v, [jnp.where(lane == 0, row, row + 1)], sv,
                    mask=lane < 2)

        gather(0, vals_a, sem_a)
        gather(1, vals_b, sem_b)

        def body(j, _):
            c0 = 2 * j
            wait(vals_a, sem_a)

            @pl.when(c0 + 2 < n_chunks)
            def _():
                gather(c0 + 2, vals_a, sem_a)

            accumulate(c0, vals_a)

            wait(vals_b, sem_b)

            @pl.when(c0 + 3 < n_chunks)
            def _():
                gather(c0 + 3, vals_b, sem_b)

            accumulate(c0 + 1, vals_b)
            return 0

        lax.fori_loop(0, n_chunks // 2, body, 0)

        out_base = pl.multiple_of(wid * rows_per_w, 8)
        pltpu.sync_copy(out_v, out_hbm.at[pl.ds(out_base, rows_per_w)])

    return sc_kernel


def kernel(x, table, W, b):
    B, L = x.shape
    V, E = table.shape
    info = plsc.get_sparse_core_info()

    t = _make_tc_matvec(V, E)(table.T, W)

    sc = _make_sc_kernel(B, L, V, info.num_cores, info.num_subcores)
    x_flat = x.reshape(-1).astype(jnp.int32)
    bias_vec = jnp.pad(b.reshape(-1).astype(jnp.float32), (0, 15))
    out = sc(x_flat, bias_vec, t)
    return out.reshape(B, 1)


# speedup vs baseline: 1.4454x; 1.1856x over previous
import functools

import jax
import jax.numpy as jnp
from jax import lax
from jax.experimental import pallas as pl
from jax.experimental.pallas import tpu as pltpu
from jax.experimental.pallas import tpu_sc as plsc

LANES = 16


def _make_tc_matvec(V, E, block_v=131072):
    grid = (V + block_v - 1) // block_v

    def body(tT_ref, w_ref, t_ref):
        w = w_ref[...]
        blk = tT_ref[...]
        t_ref[...] = jnp.sum(blk * w, axis=0)

    return pl.pallas_call(
        body,
        grid=(grid,),
        in_specs=[
            pl.BlockSpec((E, block_v), lambda i: (0, i)),
            pl.BlockSpec((E, 1), lambda i: (0, 0)),
        ],
        out_specs=pl.BlockSpec((block_v,), lambda i: (i,)),
        out_shape=jax.ShapeDtypeStruct((V,), jnp.float32),
    )


def _make_sc_kernel(B, L, V, num_cores, num_subcores, rows_per_chunk=16):
    NW = num_cores * num_subcores
    assert B % NW == 0, (B, NW)
    rows_per_w = B // NW
    toks_per_w = rows_per_w * L
    assert rows_per_w % (2 * rows_per_chunk) == 0
    chunk = rows_per_chunk * L
    n_chunks = toks_per_w // chunk
    assert chunk % 8 == 0 and toks_per_w % 8 == 0

    nfull = L // LANES
    rem = L - nfull * LANES

    mesh = plsc.VectorSubcoreMesh(core_axis_name="c", subcore_axis_name="s")

    @functools.partial(
        pl.kernel,
        out_type=jax.ShapeDtypeStruct((B,), jnp.float32),
        mesh=mesh,
        compiler_params=pltpu.CompilerParams(
            needs_layout_passes=False, use_tc_tiling_on_sc=False),
        scratch_types=[
            pltpu.VMEM((toks_per_w,), jnp.int32),
            pltpu.VMEM((chunk,), jnp.float32),
            pltpu.VMEM((chunk,), jnp.float32),
            pltpu.VMEM((16,), jnp.float32),
            pltpu.VMEM((rows_per_w,), jnp.float32),
            pltpu.VMEM_SHARED((V,), jnp.float32),
            pltpu.SemaphoreType.DMA,
            pltpu.SemaphoreType.DMA,
        ],
    )
    def sc_kernel(x_hbm, bias_hbm, t_hbm, out_hbm,
                  idx_v, vals_a, vals_b, bias_v, out_v, t_sh, sem_a, sem_b):
        wid = lax.axis_index("s") * num_cores + lax.axis_index("c")
        sid = lax.axis_index("s")

        stage = V // 8
        assert stage % 8 == 0

        @pl.when(sid < 8)
        def _():
            off = pl.multiple_of(sid * stage, 8)
            pltpu.sync_copy(t_hbm.at[pl.ds(off, stage)],
                            t_sh.at[pl.ds(off, stage)])

        tok_base = pl.multiple_of(wid * toks_per_w, 8)
        pltpu.sync_copy(x_hbm.at[pl.ds(tok_base, toks_per_w)], idx_v)
        pltpu.sync_copy(bias_hbm, bias_v)
        plsc.subcore_barrier()

        bias = bias_v[pl.ds(0, LANES)][0]
        inv_l = jnp.float32(1.0 / L)
        lane = lax.iota(jnp.int32, LANES)
        m_lo = (lane < rem).astype(jnp.float32)
        m_hi = jnp.float32(1.0) - m_lo

        def gather(c, buf, sem):
            off = pl.multiple_of(c * chunk, 8)
            return pltpu.async_copy(t_sh.at[idx_v.at[pl.ds(off, chunk)]],
                                    buf, sem)

        def wait(buf, sem):
            pltpu.make_async_copy(t_sh.at[idx_v.at[pl.ds(0, chunk)]],
                                  buf, sem).wait()

        def accumulate(c, buf):
            row0 = c * rows_per_chunk
            for p in range(rows_per_chunk // 2):
                base = p * 2 * L
                acc_a = buf[pl.ds(base, LANES)]
                for k in range(1, nfull):
                    acc_a = acc_a + buf[pl.ds(base + k * LANES, LANES)]
                vm = buf[pl.ds(base + nfull * LANES, LANES)]
                acc_b = buf[pl.ds(base + L + rem, LANES)]
                for k in range(1, nfull):
                    acc_b = acc_b + buf[pl.ds(base + L + rem + k * LANES,
                                              LANES)]
                s0 = jnp.sum(acc_a + vm * m_lo) * inv_l + bias
                s1 = jnp.sum(acc_b + vm * m_hi) * inv_l + bias
                row = row0 + 2 * p
                sv = jnp.where(lane == 0, s0, s1)
                plsc.store_scatter(
                    out_v, [jnp.where(lane == 0, row, row + 1)], sv,
                    mask=lane < 2)

        gather(0, vals_a, sem_a)
        gather(1, vals_b, sem_b)

        def body(j, _):
            c0 = 2 * j
            wait(vals_a, sem_a)

            @pl.when(c0 + 2 < n_chunks)
            def _():
                gather(c0 + 2, vals_a, sem_a)

            accumulate(c0, vals_a)

            wait(vals_b, sem_b)

            @pl.when(c0 + 3 < n_chunks)
            def _():
                gather(c0 + 3, vals_b, sem_b)

            accumulate(c0 + 1, vals_b)
            return 0

        lax.fori_loop(0, n_chunks // 2, body, 0)

        out_base = pl.multiple_of(wid * rows_per_w, 8)
        pltpu.sync_copy(out_v, out_hbm.at[pl.ds(out_base, rows_per_w)])

    return sc_kernel


def kernel(x, table, W, b):
    B, L = x.shape
    V, E = table.shape
    info = plsc.get_sparse_core_info()

    t = _make_tc_matvec(V, E)(table.T, W)

    sc = _make_sc_kernel(B, L, V, info.num_cores, info.num_subcores)
    x_flat = x.reshape(-1).astype(jnp.int32)
    bias_vec = jnp.pad(b.reshape(-1).astype(jnp.float32), (0, 15))
    out = sc(x_flat, bias_vec, t)
    return out.reshape(B, 1)
